# Initial kernel scaffold; baseline (speedup 1.0000x reference)
import jax
import jax.numpy as jnp
from jax.experimental import pallas as pl


def kernel(x, params):
    raise NotImplementedError("write your pallas kernel here")


# speedup vs baseline: 13.5743x; 13.5743x over previous
import functools

import jax
import jax.numpy as jnp
from jax import lax
from jax.experimental import pallas as pl

DIM = 192
E = 8
TOPK = 2
H = 64
W = 64
HW = H * W


def _shift(a, dh, dw):
    if dh > 0:
        a = jnp.concatenate([a[dh:], jnp.zeros((dh,) + a.shape[1:], a.dtype)], axis=0)
    elif dh < 0:
        a = jnp.concatenate([jnp.zeros((-dh,) + a.shape[1:], a.dtype), a[:dh]], axis=0)
    if dw > 0:
        a = jnp.concatenate([a[:, dw:], jnp.zeros(a.shape[:1] + (dw,) + a.shape[2:], a.dtype)], axis=1)
    elif dw < 0:
        a = jnp.concatenate([jnp.zeros(a.shape[:1] + (-dw,) + a.shape[2:], a.dtype), a[:, :dw]], axis=1)
    return a


def _matmul_ct(a, w):
    return lax.dot_general(a, w, (((1,), (1,)), ((), ())),
                           preferred_element_type=jnp.float32)


def _moe_step(x_ref, rw_ref, rb_ref, eb_ref, weff_ref, bias_ref,
              dw_ref, dwb_ref, fc1_ref, fc1b_ref, fc2_ref, fc2b_ref,
              out_ref):
    xb = x_ref[0]
    xf = xb.reshape(HW, DIM)

    gvec = jnp.mean(xf, axis=0, keepdims=True)
    logits = _matmul_ct(gvec, rw_ref[...]) + rb_ref[...]
    logits = jnp.clip(logits, -10.0, 10.0) + eb_ref[...]
    m = jnp.max(logits)
    p = jnp.exp(logits - m)
    probs = p / jnp.sum(p)
    probs = jnp.clip(probs, 1e-6, 1.0)

    iota = lax.broadcasted_iota(jnp.int32, (1, E), 1)
    v1 = jnp.max(probs)
    i1 = jnp.min(jnp.where(probs == v1, iota, E))
    sel1 = iota == i1
    rest = jnp.where(sel1, -jnp.inf, probs)
    v2 = jnp.max(rest)
    i2 = jnp.min(jnp.where((rest == v2) & (~sel1), iota, E))
    sel2 = iota == i2
    denom = v1 + v2 + 1e-8
    w1 = v1 / denom
    w2 = v2 / denom
    gates = jnp.where(sel1, w1, 0.0) + jnp.where(sel2, w2, 0.0)

    def gate(e):
        return jnp.sum(jnp.where(iota == e, gates, 0.0))

    def s_vec(j):
        h = jnp.maximum(_matmul_ct(gvec, fc1_ref[j]) + fc1b_ref[j][None, :], 0.0)
        return jax.nn.sigmoid(_matmul_ct(h, fc2_ref[j]) + fc2b_ref[j][None, :])

    alpha = (w1 + w2) + gate(3) * s_vec(0) + gate(7) * s_vec(1)

    lap = (_shift(xb, -1, 0) + _shift(xb, 1, 0) +
           _shift(xb, 0, -1) + _shift(xb, 0, 1) - 4.0 * xb)

    def dwconv_gelu(j):
        acc = jnp.broadcast_to(dwb_ref[j][None, None, :], (H, W, DIM))
        for a in range(3):
            for c in range(3):
                acc = acc + _shift(xb, a - 1, c - 1) * dw_ref[j, a * 3 + c][None, None, :]
        return jax.nn.gelu(acc)

    u0 = dwconv_gelu(0)
    u4 = dwconv_gelu(1)

    w_x = gate(1) * weff_ref[1] + gate(5) * weff_ref[5]
    w_lap = gate(2) * weff_ref[2] + gate(6) * weff_ref[6]
    w_u0 = gate(0) * weff_ref[0]
    w_u4 = gate(4) * weff_ref[4]

    y = (_matmul_ct(xf, w_x) +
         _matmul_ct(lap.reshape(HW, DIM), w_lap) +
         _matmul_ct(u0.reshape(HW, DIM), w_u0) +
         _matmul_ct(u4.reshape(HW, DIM), w_u4))

    btot = lax.dot_general(gates, bias_ref[...], (((1,), (0,)), ((), ())),
                           preferred_element_type=jnp.float32)

    out = xf * alpha + y + btot
    out_ref[0] = out.reshape(H, W, DIM)


def kernel(x, params):
    B = x.shape[0]
    xh = jnp.transpose(x, (0, 2, 3, 1))

    weff = []
    bias = []
    for e in range(E):
        t = e % 4
        if t == 3:
            weff.append(jnp.zeros((DIM, DIM), jnp.float32))
            bias.append(jnp.zeros((DIM,), jnp.float32))
        else:
            w = params[f'e{e}_pw_w'].reshape(DIM, DIM)
            if t == 1:
                w = w * params[f'e{e}_gain'][None, :]
            weff.append(w)
            bias.append(params[f'e{e}_pw_b'])
    weff = jnp.stack(weff)
    bias = jnp.stack(bias)

    dw = jnp.stack([params['e0_dw_w'].reshape(DIM, 9).T,
                    params['e4_dw_w'].reshape(DIM, 9).T])
    dwb = jnp.stack([params['e0_dw_b'], params['e4_dw_b']])
    fc1 = jnp.stack([params['e3_fc1_w'], params['e7_fc1_w']])
    fc1b = jnp.stack([params['e3_fc1_b'], params['e7_fc1_b']])
    fc2 = jnp.stack([params['e3_fc2_w'], params['e7_fc2_w']])
    fc2b = jnp.stack([params['e3_fc2_b'], params['e7_fc2_b']])

    rw = params['router_w']
    rb = params['router_b'][None, :]
    eb = params['expert_bias'][None, :]

    full = lambda s: pl.BlockSpec(s, lambda b: (0,) * len(s))
    out_h = pl.pallas_call(
        _moe_step,
        grid=(B,),
        in_specs=[
            pl.BlockSpec((1, H, W, DIM), lambda b: (b, 0, 0, 0)),
            full(rw.shape), full(rb.shape), full(eb.shape),
            full(weff.shape), full(bias.shape),
            full(dw.shape), full(dwb.shape),
            full(fc1.shape), full(fc1b.shape),
            full(fc2.shape), full(fc2b.shape),
        ],
        out_specs=pl.BlockSpec((1, H, W, DIM), lambda b: (b, 0, 0, 0)),
        out_shape=jax.ShapeDtypeStruct((B, H, W, DIM), jnp.float32),
    )(xh, rw, rb, eb, weff, bias, dw, dwb, fc1, fc1b, fc2, fc2b)

    out = jnp.transpose(out_h, (0, 3, 1, 2))
    return (out, jnp.array(0.0, dtype=x.dtype))


# speedup vs baseline: 22.2164x; 1.6367x over previous
import jax
import jax.numpy as jnp
from jax import lax
from jax.experimental import pallas as pl

DIM = 192
E = 8
H = 64
W = 64
HW = H * W


def _shift(a, dh, dw):
    if dh > 0:
        a = jnp.concatenate([a[dh:], jnp.zeros((dh,) + a.shape[1:], a.dtype)], axis=0)
    elif dh < 0:
        a = jnp.concatenate([jnp.zeros((-dh,) + a.shape[1:], a.dtype), a[:dh]], axis=0)
    if dw > 0:
        a = jnp.concatenate([a[:, dw:], jnp.zeros(a.shape[:1] + (dw,) + a.shape[2:], a.dtype)], axis=1)
    elif dw < 0:
        a = jnp.concatenate([jnp.zeros(a.shape[:1] + (-dw,) + a.shape[2:], a.dtype), a[:, :dw]], axis=1)
    return a


def _matmul_ct(a, w):
    return lax.dot_general(a, w, (((1,), (1,)), ((), ())),
                           preferred_element_type=jnp.float32)


def _moe_step(x_ref, rw_ref, rb_ref, eb_ref,
              w0_ref, b0_ref, dw0_ref, db0_ref,
              w1_ref, b1_ref, g1_ref,
              w2_ref, b2_ref,
              f3a_ref, f3ab_ref, f3b_ref, f3bb_ref,
              w4_ref, b4_ref, dw4_ref, db4_ref,
              w5_ref, b5_ref, g5_ref,
              w6_ref, b6_ref,
              f7a_ref, f7ab_ref, f7b_ref, f7bb_ref,
              out_ref):
    xb = x_ref[0]
    xf = xb.reshape(HW, DIM)

    gvec = jnp.mean(xf, axis=0, keepdims=True)
    logits = _matmul_ct(gvec, rw_ref[...]) + rb_ref[...]
    logits = jnp.clip(logits, -10.0, 10.0) + eb_ref[...]
    m = jnp.max(logits)
    p = jnp.exp(logits - m)
    probs = p / jnp.sum(p)
    probs = jnp.clip(probs, 1e-6, 1.0)

    iota = lax.broadcasted_iota(jnp.int32, (1, E), 1)
    v1 = jnp.max(probs)
    i1 = jnp.min(jnp.where(probs == v1, iota, E))
    sel1 = iota == i1
    rest = jnp.where(sel1, -jnp.inf, probs)
    v2 = jnp.max(rest)
    i2 = jnp.min(jnp.where((rest == v2) & (~sel1), iota, E))
    sel2 = iota == i2
    denom = v1 + v2 + 1e-8
    wa = v1 / denom
    wb = v2 / denom
    gates = jnp.where(sel1, wa, 0.0) + jnp.where(sel2, wb, 0.0)

    def gate(e):
        return jnp.sum(jnp.where(iota == e, gates, 0.0))

    g0, g1, g2, g3 = gate(0), gate(1), gate(2), gate(3)
    g4, g5, g6, g7 = gate(4), gate(5), gate(6), gate(7)

    def s_vec(fa, fab, fb, fbb):
        h = jnp.maximum(_matmul_ct(gvec, fa[...]) + fab[...], 0.0)
        return jax.nn.sigmoid(_matmul_ct(h, fb[...]) + fbb[...])

    alpha = ((wa + wb)
             + g3 * s_vec(f3a_ref, f3ab_ref, f3b_ref, f3bb_ref)
             + g7 * s_vec(f7a_ref, f7ab_ref, f7b_ref, f7bb_ref))

    btot = (g0 * b0_ref[...] + g1 * b1_ref[...] + g2 * b2_ref[...]
            + g4 * b4_ref[...] + g5 * b5_ref[...] + g6 * b6_ref[...])

    out_ref[0] = (xf * alpha + btot).reshape(H, W, DIM)

    @pl.when(g1 + g5 > 0.0)
    def _freq():
        wfr = g1 * (w1_ref[...] * g1_ref[...]) + g5 * (w5_ref[...] * g5_ref[...])
        out_ref[0] += _matmul_ct(xf, wfr).reshape(H, W, DIM)

    @pl.when(g2 + g6 > 0.0)
    def _edge():
        lap = (_shift(xb, -1, 0) + _shift(xb, 1, 0) +
               _shift(xb, 0, -1) + _shift(xb, 0, 1) - 4.0 * xb)
        wed = g2 * w2_ref[...] + g6 * w6_ref[...]
        out_ref[0] += _matmul_ct(lap.reshape(HW, DIM), wed).reshape(H, W, DIM)

    def texture(dw_ref, db_ref, w_ref, g):
        acc = jnp.broadcast_to(db_ref[...][None], (H, W, DIM))
        for a in range(3):
            for c in range(3):
                acc = acc + _shift(xb, a - 1, c - 1) * dw_ref[a * 3 + c][None, None, :]
        u = jax.nn.gelu(acc)
        out_ref[0] += _matmul_ct(u.reshape(HW, DIM), g * w_ref[...]).reshape(H, W, DIM)

    @pl.when(g0 > 0.0)
    def _tex0():
        texture(dw0_ref, db0_ref, w0_ref, g0)

    @pl.when(g4 > 0.0)
    def _tex4():
        texture(dw4_ref, db4_ref, w4_ref, g4)


def kernel(x, params):
    B = x.shape[0]
    xh = jnp.transpose(x, (0, 2, 3, 1))

    def pw(e):
        return params[f'e{e}_pw_w'].reshape(DIM, DIM)

    def row(v):
        return v[None, :]

    dw0 = params['e0_dw_w'].reshape(DIM, 9).T
    dw4 = params['e4_dw_w'].reshape(DIM, 9).T

    operands = [
        xh,
        params['router_w'], row(params['router_b']), row(params['expert_bias']),
        pw(0), row(params['e0_pw_b']), dw0, row(params['e0_dw_b']),
        pw(1), row(params['e1_pw_b']), row(params['e1_gain']),
        pw(2), row(params['e2_pw_b']),
        params['e3_fc1_w'], row(params['e3_fc1_b']),
        params['e3_fc2_w'], row(params['e3_fc2_b']),
        pw(4), row(params['e4_pw_b']), dw4, row(params['e4_dw_b']),
        pw(5), row(params['e5_pw_b']), row(params['e5_gain']),
        pw(6), row(params['e6_pw_b']),
        params['e7_fc1_w'], row(params['e7_fc1_b']),
        params['e7_fc2_w'], row(params['e7_fc2_b']),
    ]

    full = lambda a: pl.BlockSpec(a.shape, lambda b: (0,) * a.ndim)
    in_specs = [pl.BlockSpec((1, H, W, DIM), lambda b: (b, 0, 0, 0))]
    in_specs += [full(a) for a in operands[1:]]

    out_h = pl.pallas_call(
        _moe_step,
        grid=(B,),
        in_specs=in_specs,
        out_specs=pl.BlockSpec((1, H, W, DIM), lambda b: (b, 0, 0, 0)),
        out_shape=jax.ShapeDtypeStruct((B, H, W, DIM), jnp.float32),
    )(*operands)

    out = jnp.transpose(out_h, (0, 3, 1, 2))
    return (out, jnp.array(0.0, dtype=x.dtype))
